# baseline (device time: 17448 ns/iter reference)
import jax
import jax.numpy as jnp
from jax import lax
from jax.experimental import pallas as pl
from jax.experimental.pallas import tpu as pltpu

N_DEV = 4


def kernel(x, router_W, route_idx, expert_W):
    n_tok, d_model = x.shape
    e_loc, _, d_out = expert_W.shape
    n_exp = router_W.shape[1]
    m_blk = n_tok // N_DEV

    def body(x_ref, rw_ref, idx_ref, ew_hbm, out_ref,
             ew_f32, ew_b16, scores_ref, send_ref, comm_ref,
             send_sems, recv_sems, in_sems):
        my = lax.axis_index("i")

        cp_ew = pltpu.make_async_copy(ew_hbm, ew_f32, in_sems.at[0])
        cp_ew.start()

        barrier_sem = pltpu.get_barrier_semaphore()
        for k in range(1, N_DEV):
            peer = lax.rem(my + k, N_DEV)
            pl.semaphore_signal(
                barrier_sem, inc=1,
                device_id=(peer,), device_id_type=pl.DeviceIdType.MESH,
            )

        scores_ref[:, :] = jnp.dot(x_ref[:, :], rw_ref[:, :],
                                   preferred_element_type=jnp.float32)

        cp_ew.wait()
        for e in range(e_loc):
            ew_b16[e] = ew_f32[e].astype(jnp.bfloat16)
        ew_b = ew_b16[:, :, :].reshape(e_loc * d_model, d_out)

        def block_partial(row0):
            xb = x_ref[pl.ds(row0, m_blk), :]
            sb = scores_ref[pl.ds(row0, m_blk), :]
            i0 = idx_ref[pl.ds(row0, m_blk), 0:1]
            i1 = idx_ref[pl.ds(row0, m_blk), 1:2]
            eio = lax.broadcasted_iota(jnp.int32, (m_blk, n_exp), 1)
            s0 = jnp.sum(jnp.where(i0 == eio, sb, 0.0), axis=1, keepdims=True)
            s1 = jnp.sum(jnp.where(i1 == eio, sb, 0.0), axis=1, keepdims=True)
            w0 = 1.0 / (1.0 + jnp.exp(s1 - s0))
            w1 = 1.0 - w0
            xs = []
            for e in range(e_loc):
                ge = my * e_loc + e
                coef = jnp.where(i0 == ge, w0, 0.0) + jnp.where(i1 == ge, w1, 0.0)
                xs.append((xb * coef).astype(jnp.bfloat16))
            xcat = jnp.concatenate(xs, axis=1)
            return jnp.dot(xcat, ew_b,
                           preferred_element_type=jnp.float32)

        order = (2, 1, 3)
        rdmas = []
        for j, k in enumerate(order):
            dst = lax.rem(my + k, N_DEV)
            send_ref[k - 1] = block_partial(dst * m_blk).astype(jnp.bfloat16)
            if j == 0:
                pl.semaphore_wait(barrier_sem, N_DEV - 1)
            rdma = pltpu.make_async_remote_copy(
                src_ref=send_ref.at[k - 1],
                dst_ref=comm_ref.at[k - 1],
                send_sem=send_sems.at[k - 1],
                recv_sem=recv_sems.at[k - 1],
                device_id=(dst,),
                device_id_type=pl.DeviceIdType.MESH,
            )
            rdma.start()
            rdmas.append(rdma)

        acc = block_partial(my * m_blk)
        for j, k in enumerate(order):
            rdmas[j].wait()
            acc = acc + comm_ref[k - 1].astype(jnp.float32)
        out_ref[:, :] = acc.astype(jnp.bfloat16)

    f = pl.pallas_call(
        body,
        out_shape=jax.ShapeDtypeStruct((m_blk, d_out), jnp.bfloat16),
        in_specs=[
            pl.BlockSpec(memory_space=pltpu.VMEM),
            pl.BlockSpec(memory_space=pltpu.VMEM),
            pl.BlockSpec(memory_space=pltpu.VMEM),
            pl.BlockSpec(memory_space=pl.ANY),
        ],
        out_specs=pl.BlockSpec(memory_space=pltpu.VMEM),
        scratch_shapes=[
            pltpu.VMEM((e_loc, d_model, d_out), jnp.float32),
            pltpu.VMEM((e_loc, d_model, d_out), jnp.bfloat16),
            pltpu.VMEM((n_tok, n_exp), jnp.float32),
            pltpu.VMEM((N_DEV - 1, m_blk, d_out), jnp.bfloat16),
            pltpu.VMEM((N_DEV - 1, m_blk, d_out), jnp.bfloat16),
            pltpu.SemaphoreType.DMA((N_DEV - 1,)),
            pltpu.SemaphoreType.DMA((N_DEV - 1,)),
            pltpu.SemaphoreType.DMA((1,)),
        ],
        compiler_params=pltpu.CompilerParams(collective_id=0),
    )
    return f(x, router_W, route_idx, expert_W)


# device time: 15816 ns/iter; 1.1032x vs baseline; 1.1032x over previous
import jax
import jax.numpy as jnp
from jax import lax
from jax.experimental import pallas as pl
from jax.experimental.pallas import tpu as pltpu

N_DEV = 4


def kernel(x, router_W, route_idx, expert_W):
    n_tok, d_model = x.shape
    e_loc, _, d_out = expert_W.shape
    n_exp = router_W.shape[1]
    m_blk = n_tok // N_DEV

    def body(x_ref, rw_ref, idx_ref, ew_ref, out_ref,
             scores_ref, send_ref, comm_ref, send_sems, recv_sems):
        my = lax.axis_index("i")

        barrier_sem = pltpu.get_barrier_semaphore()
        for k in range(1, N_DEV):
            peer = lax.rem(my + k, N_DEV)
            pl.semaphore_signal(
                barrier_sem, inc=1,
                device_id=(peer,), device_id_type=pl.DeviceIdType.MESH,
            )

        scores_ref[:, :] = jnp.dot(x_ref[:, :], rw_ref[:, :],
                                   preferred_element_type=jnp.float32)

        ew_b = ew_ref[:, :, :].reshape(e_loc * d_model, d_out)

        def block_partial(row0, rows):
            xb = x_ref[pl.ds(row0, rows), :]
            sb = scores_ref[pl.ds(row0, rows), :]
            i0 = idx_ref[pl.ds(row0, rows), 0:1]
            i1 = idx_ref[pl.ds(row0, rows), 1:2]
            eio = lax.broadcasted_iota(jnp.int32, (rows, n_exp), 1)
            s0 = jnp.sum(jnp.where(i0 == eio, sb, 0.0), axis=1, keepdims=True)
            s1 = jnp.sum(jnp.where(i1 == eio, sb, 0.0), axis=1, keepdims=True)
            w0 = 1.0 / (1.0 + jnp.exp(s1 - s0))
            w1 = 1.0 - w0
            xs = []
            for e in range(e_loc):
                ge = my * e_loc + e
                coef = jnp.where(i0 == ge, w0, 0.0) + jnp.where(i1 == ge, w1, 0.0)
                xs.append((xb * coef).astype(jnp.bfloat16))
            xcat = jnp.concatenate(xs, axis=1)
            return jnp.dot(xcat, ew_b,
                           preferred_element_type=jnp.float32)

        m_half = m_blk // 2
        order = (2, 1, 3)
        rdmas = []
        for j, k in enumerate(order):
            dst = lax.rem(my + k, N_DEV)
            for h in range(2):
                send_ref[k - 1, pl.ds(h * m_half, m_half)] = (
                    block_partial(dst * m_blk + h * m_half, m_half)
                    .astype(jnp.bfloat16)
                )
                if j == 0 and h == 0:
                    pl.semaphore_wait(barrier_sem, N_DEV - 1)
                rdma = pltpu.make_async_remote_copy(
                    src_ref=send_ref.at[k - 1, pl.ds(h * m_half, m_half)],
                    dst_ref=comm_ref.at[k - 1, pl.ds(h * m_half, m_half)],
                    send_sem=send_sems.at[k - 1, h],
                    recv_sem=recv_sems.at[k - 1, h],
                    device_id=(dst,),
                    device_id_type=pl.DeviceIdType.MESH,
                )
                rdma.start()
                rdmas.append(rdma)

        acc = block_partial(my * m_blk, m_blk)
        for j, k in enumerate(order):
            rdmas[2 * j].wait()
            rdmas[2 * j + 1].wait()
            acc = acc + comm_ref[k - 1].astype(jnp.float32)
        out_ref[:, :] = acc.astype(jnp.bfloat16)

    f = pl.pallas_call(
        body,
        out_shape=jax.ShapeDtypeStruct((m_blk, d_out), jnp.bfloat16),
        in_specs=[pl.BlockSpec(memory_space=pltpu.VMEM)] * 4,
        out_specs=pl.BlockSpec(memory_space=pltpu.VMEM),
        scratch_shapes=[
            pltpu.VMEM((n_tok, n_exp), jnp.float32),
            pltpu.VMEM((N_DEV - 1, m_blk, d_out), jnp.bfloat16),
            pltpu.VMEM((N_DEV - 1, m_blk, d_out), jnp.bfloat16),
            pltpu.SemaphoreType.DMA((N_DEV - 1, 2)),
            pltpu.SemaphoreType.DMA((N_DEV - 1, 2)),
        ],
        compiler_params=pltpu.CompilerParams(collective_id=0),
    )
    return f(x, router_W, route_idx, expert_W.astype(jnp.bfloat16))
